# Initial kernel scaffold; baseline (speedup 1.0000x reference)
import jax
import jax.numpy as jnp
from jax.experimental import pallas as pl


def kernel(x, W):
    raise NotImplementedError("write your pallas kernel here")


# speedup vs baseline: 1.4017x; 1.4017x over previous
import functools

import jax
import jax.numpy as jnp
from jax.experimental import pallas as pl
from jax.experimental.pallas import tpu as pltpu

_CAPF = 1.25
_K = 8
_BT = 256


def _router_kernel(x_ref, w_ref,
                   idx_ref, wts_ref, mask_ref,
                   counts_ref, avgp_ref, ent_ref, gini_ref, ctr_ref, drop_ref,
                   acc_counts, acc_probs, acc_ent, acc_keep, acc_drop,
                   *, bt, e, k, nt, cap):
    i = pl.program_id(0)
    nsteps = pl.num_programs(0)

    @pl.when(i == 0)
    def _init():
        acc_counts[...] = jnp.zeros_like(acc_counts)
        acc_probs[...] = jnp.zeros_like(acc_probs)
        acc_ent[...] = jnp.zeros_like(acc_ent)
        acc_keep[...] = jnp.zeros_like(acc_keep)
        acc_drop[...] = jnp.zeros_like(acc_drop)

    xb = x_ref[...]
    logits = jax.lax.dot_general(xb, w_ref[...], (((1,), (1,)), ((), ())),
                                 preferred_element_type=jnp.float32)
    m = jnp.max(logits, axis=-1, keepdims=True)
    el = jnp.exp(logits - m)
    probs = el / jnp.sum(el, axis=-1, keepdims=True)

    lane = jax.lax.broadcasted_iota(jnp.int32, (bt, e), 1)
    cur = probs
    iks, vks = [], []
    for _ in range(k):
        mk = jnp.max(cur, axis=-1, keepdims=True)
        ik = jnp.min(jnp.where(cur == mk, lane, e), axis=-1, keepdims=True)
        iks.append(ik)
        vks.append(mk)
        cur = jnp.where(lane == ik, -jnp.inf, cur)
    tidx = jnp.concatenate(iks, axis=1)
    tvals = jnp.concatenate(vks, axis=1)
    tw = tvals / jnp.sum(tvals, axis=1, keepdims=True)

    h = jnp.zeros((bt, e), jnp.float32)
    for kk in range(k):
        h = h + (lane == iks[kk]).astype(jnp.float32)

    r2 = jax.lax.broadcasted_iota(jnp.int32, (bt, bt), 0)
    c2 = jax.lax.broadcasted_iota(jnp.int32, (bt, bt), 1)
    tri = (c2 < r2).astype(jnp.float32)
    excl = jax.lax.dot_general(tri, h, (((1,), (0,)), ((), ())),
                               preferred_element_type=jnp.float32)
    excl = excl + acc_counts[...]

    keeps = []
    for kk in range(k):
        posk = jnp.sum(jnp.where(lane == iks[kk], excl, 0.0),
                       axis=1, keepdims=True)
        keeps.append((posk < cap).astype(jnp.float32))
    maskb = jnp.concatenate(keeps, axis=1)

    idx_ref[...] = tidx
    masksum = jnp.sum(maskb, axis=1, keepdims=True)
    wts_ref[...] = tw * maskb / (masksum + 1e-10)
    mask_ref[...] = maskb

    kept = jnp.zeros((1, e), jnp.float32)
    for kk in range(k):
        sel = jnp.where((lane == iks[kk]) & (keeps[kk] > 0.0), 1.0, 0.0)
        kept = kept + jnp.sum(sel, axis=0, keepdims=True)

    acc_counts[...] = acc_counts[...] + jnp.sum(h, axis=0, keepdims=True)
    acc_probs[...] = acc_probs[...] + jnp.sum(probs, axis=0, keepdims=True)
    acc_ent[...] = acc_ent[...] + jnp.sum(-probs * jnp.log(probs + 1e-10),
                                          keepdims=True)
    acc_keep[...] = acc_keep[...] + kept
    acc_drop[...] = acc_drop[...] + jnp.sum(1.0 - maskb, keepdims=True)

    @pl.when(i == nsteps - 1)
    def _finalize():
        cnt = acc_counts[...]
        counts_ref[...] = cnt
        avgp_ref[...] = acc_probs[...] / nt
        ent_ref[...] = acc_ent[...] / nt
        cb = jnp.broadcast_to(cnt, (e, e))
        rr = jax.lax.broadcasted_iota(jnp.int32, (e, e), 0)
        cc = jax.lax.broadcasted_iota(jnp.int32, (e, e), 1)
        ccol = jnp.sum(jnp.where(rr == cc, cb, 0.0), axis=1, keepdims=True)
        less = jnp.sum((cb < ccol).astype(jnp.float32), axis=1, keepdims=True)
        eq = jnp.sum((cb == ccol).astype(jnp.float32), axis=1, keepdims=True)
        num = jnp.sum(ccol * (2.0 * less + eq - e), keepdims=True)
        tot = jnp.sum(cnt, keepdims=True)
        gini_ref[...] = num / (e * tot + 1e-10)
        ctr_ref[...] = acc_keep[...].astype(jnp.int32)
        drop_ref[...] = acc_drop[...]


@jax.jit
def kernel(x, W):
    nt, hidden = x.shape
    e = W.shape[0]
    k = _K
    bt = _BT
    cap = int(nt * k / e * _CAPF)
    grid = nt // bt
    kfn = functools.partial(_router_kernel, bt=bt, e=e, k=k, nt=nt, cap=cap)
    outs = pl.pallas_call(
        kfn,
        grid=(grid,),
        in_specs=[
            pl.BlockSpec((bt, hidden), lambda i: (i, 0)),
            pl.BlockSpec((e, hidden), lambda i: (0, 0)),
        ],
        out_specs=[
            pl.BlockSpec((bt, k), lambda i: (i, 0)),
            pl.BlockSpec((bt, k), lambda i: (i, 0)),
            pl.BlockSpec((bt, k), lambda i: (i, 0)),
            pl.BlockSpec((1, e), lambda i: (0, 0)),
            pl.BlockSpec((1, e), lambda i: (0, 0)),
            pl.BlockSpec((1, 1), lambda i: (0, 0)),
            pl.BlockSpec((1, 1), lambda i: (0, 0)),
            pl.BlockSpec((1, e), lambda i: (0, 0)),
            pl.BlockSpec((1, 1), lambda i: (0, 0)),
        ],
        out_shape=[
            jax.ShapeDtypeStruct((nt, k), jnp.int32),
            jax.ShapeDtypeStruct((nt, k), jnp.float32),
            jax.ShapeDtypeStruct((nt, k), jnp.float32),
            jax.ShapeDtypeStruct((1, e), jnp.float32),
            jax.ShapeDtypeStruct((1, e), jnp.float32),
            jax.ShapeDtypeStruct((1, 1), jnp.float32),
            jax.ShapeDtypeStruct((1, 1), jnp.float32),
            jax.ShapeDtypeStruct((1, e), jnp.int32),
            jax.ShapeDtypeStruct((1, 1), jnp.float32),
        ],
        scratch_shapes=[
            pltpu.VMEM((1, e), jnp.float32),
            pltpu.VMEM((1, e), jnp.float32),
            pltpu.VMEM((1, 1), jnp.float32),
            pltpu.VMEM((1, e), jnp.float32),
            pltpu.VMEM((1, 1), jnp.float32),
        ],
    )(x, W)
    tidx, tw, maskb, counts, avgp, ent, gini, ctr, drop = outs
    return (tidx, tw, maskb,
            counts.reshape(e), avgp.reshape(e),
            ent.reshape(()), gini.reshape(()),
            ctr.reshape(e), drop.reshape(()))
